# baseline (speedup 1.0000x reference)
import functools

import jax
import jax.numpy as jnp
from jax.experimental import pallas as pl
from jax.experimental.pallas import tpu as pltpu

N = 10000
E = 320000
D = 128
DE = 16
NL = 3
DELTA = 2.5


def _mm_k(a_ref, b_ref, o_ref):
    o_ref[...] = jnp.dot(a_ref[...], b_ref[...], preferred_element_type=jnp.float32)


def _mm(a, b, block_rows=None):
    m, k = a.shape
    k2, n = b.shape
    if block_rows is None:
        block_rows = m
    grid = (m // block_rows,)
    return pl.pallas_call(
        _mm_k,
        grid=grid,
        in_specs=[
            pl.BlockSpec((block_rows, k), lambda i: (i, 0)),
            pl.BlockSpec((k, n), lambda i: (0, 0)),
        ],
        out_specs=pl.BlockSpec((block_rows, n), lambda i: (i, 0)),
        out_shape=jax.ShapeDtypeStruct((m, n), jnp.float32),
    )(a, b)


def kernel(h, e, edge_index, W_h, b_h, W_e, b_e, pre_Ws, pre_bs, post_Ws, post_bs, Wr0, br0, Wr1, br1, Wr2, br2):
    src = edge_index[0]
    dst = edge_index[1]

    x = _mm(h, W_h, block_rows=2000) + b_h

    deg = jax.ops.segment_sum(jnp.ones((E,), jnp.float32), dst, num_segments=N)
    degc = jnp.maximum(deg, 1.0)
    logd = jnp.log(deg + 1.0)
    amp = logd / DELTA
    att = DELTA / jnp.maximum(logd, 1e-6)
    has = (deg > 0)[:, None]

    for l in range(NL):
        Wpre = pre_Ws[l]
        Ws, We2, Wd = Wpre[:D], Wpre[D:2 * D], Wpre[2 * D:]
        Me = W_e @ We2
        be = b_e @ We2 + pre_bs[l]
        Psrc = _mm(x, Ws, block_rows=2000)
        Pdst = _mm(x, Wd, block_rows=2000)
        Et = _mm(e, Me, block_rows=8000) + be

        m = jax.nn.relu(Psrc[src] + Pdst[dst] + Et)

        s = jax.ops.segment_sum(m, dst, num_segments=N)
        mean = s / degc[:, None]
        mx = jnp.where(has, jax.ops.segment_max(m, dst, num_segments=N), 0.0)
        if l < NL - 1:
            mn = jnp.where(has, -jax.ops.segment_max(-m, dst, num_segments=N), 0.0)
            q = jax.ops.segment_sum(m * m, dst, num_segments=N) / degc[:, None]
            std = jnp.sqrt(jax.nn.relu(q - mean * mean) + 1e-5)
            agg = jnp.concatenate([mean, mx, mn, std], axis=1)
            feats = jnp.concatenate([x, agg, agg * amp[:, None], agg * att[:, None]], axis=1)
        else:
            agg = jnp.concatenate([mean, mx, s], axis=1)
            feats = jnp.concatenate([x, agg, agg * amp[:, None]], axis=1)
        x = x + _mm(feats, post_Ws[l], block_rows=1000) + post_bs[l]

    hg = jnp.mean(x, axis=0, keepdims=True)
    r = jax.nn.relu(hg @ Wr0 + br0)
    r = jax.nn.relu(r @ Wr1 + br1)
    return r @ Wr2 + br2
